# Initial kernel scaffold; baseline (speedup 1.0000x reference)
import functools

import jax
import jax.numpy as jnp
from jax import lax
from jax.experimental import pallas as pl
from jax.experimental.pallas import tpu as pltpu
from jax.experimental.pallas import tpu_sc as plsc

B = 16384
D = 64
K = 5
NC = 2
NS = 16
NW = NC * NS
PW = B // NW
CH = 128
NCH = PW // CH
L = 16
G = CH // L


def _sc_scores(idx_all, u_weight, v_weight):
    mesh = plsc.VectorSubcoreMesh(core_axis_name="c", subcore_axis_name="s")

    @functools.partial(
        pl.kernel,
        out_type=jax.ShapeDtypeStruct((NW, 1 + K, PW), jnp.float32),
        mesh=mesh,
        scratch_types=[
            pltpu.VMEM((2 + K, PW), jnp.int32),
            pltpu.VMEM((CH, D), jnp.float32),
            pltpu.VMEM((CH, D), jnp.float32),
            pltpu.VMEM((K * CH, D), jnp.float32),
            pltpu.VMEM((1 + K, PW), jnp.float32),
            pltpu.SemaphoreType.DMA,
            pltpu.SemaphoreType.DMA,
            pltpu.SemaphoreType.DMA,
        ],
    )
    def body(idx_hbm, uw_hbm, vw_hbm, out_hbm, idx_v, ru, rv, rn, sbuf, semu, semv, semn):
        wid = lax.axis_index("s") * NC + lax.axis_index("c")
        pltpu.sync_copy(idx_hbm.at[wid], idx_v)
        iota = lax.iota(jnp.int32, L)
        for c in range(NCH):
            cu = pltpu.async_copy(uw_hbm.at[idx_v.at[0, pl.ds(c * CH, CH)]], ru, semu)
            cv = pltpu.async_copy(vw_hbm.at[idx_v.at[1, pl.ds(c * CH, CH)]], rv, semv)
            cns = [
                pltpu.async_copy(
                    vw_hbm.at[idx_v.at[2 + k, pl.ds(c * CH, CH)]],
                    rn.at[pl.ds(k * CH, CH)], semn)
                for k in range(K)
            ]
            cu.wait()
            cv.wait()
            for cn in cns:
                cn.wait()
            for g in range(G):
                rows = g * L + iota
                zero = jnp.zeros((L,), jnp.float32)

                def dbody(dd, accs, rows=rows):
                    dvec = jnp.zeros((L,), jnp.int32) + dd
                    u = plsc.load_gather(ru, [rows, dvec])
                    v = plsc.load_gather(rv, [rows, dvec])
                    s0 = accs[0] + u * v
                    ss = [
                        accs[1 + k]
                        + u * plsc.load_gather(rn, [k * CH + rows, dvec])
                        for k in range(K)
                    ]
                    return (s0, *ss)

                accs = lax.fori_loop(0, D, dbody, (zero,) * (1 + K))
                off = c * CH + g * L
                sbuf[0, pl.ds(off, L)] = -accs[0]
                for k in range(K):
                    sbuf[1 + k, pl.ds(off, L)] = accs[1 + k]
        pltpu.sync_copy(sbuf, out_hbm.at[wid])

    return body(idx_all, u_weight, v_weight)


def _loss_body(x_ref, o_ref):
    x = jnp.clip(x_ref[...], -10.0, 10.0)
    o_ref[0, 0] = jnp.sum(jnp.log1p(jnp.exp(x))) * (1.0 / B)


def kernel(pos_u, pos_v, neg_v, u_weight, v_weight):
    idx_all = jnp.concatenate(
        [pos_u[None, :], pos_v[None, :], neg_v.T.astype(jnp.int32)], axis=0)
    idx_all = idx_all.reshape(2 + K, NW, PW).transpose(1, 0, 2)
    scores = _sc_scores(idx_all, u_weight, v_weight)
    loss = pl.pallas_call(
        _loss_body,
        out_shape=jax.ShapeDtypeStruct((1, 1), jnp.float32),
    )(scores.reshape(NW * (1 + K) * PW // 2048, 2048))
    return loss[0, 0]


# speedup vs baseline: 1.5914x; 1.5914x over previous
import functools

import jax
import jax.numpy as jnp
from jax import lax
from jax.experimental import pallas as pl
from jax.experimental.pallas import tpu as pltpu
from jax.experimental.pallas import tpu_sc as plsc

B = 16384
D = 64
K = 5
NC = 2
NS = 16
NW = NC * NS
PW = B // NW
CH = 128
NCH = PW // CH
L = 16
G = CH // L


def _sc_scores(idx_all, u_weight, v_weight):
    mesh = plsc.VectorSubcoreMesh(core_axis_name="c", subcore_axis_name="s")

    @functools.partial(
        pl.kernel,
        out_type=jax.ShapeDtypeStruct((NW, 1 + K, PW), jnp.float32),
        mesh=mesh,
        scratch_types=[
            pltpu.VMEM(((2 + K) * PW,), jnp.int32),
            pltpu.VMEM((CH, D), jnp.float32),
            pltpu.VMEM((CH, D), jnp.float32),
            pltpu.VMEM((K * CH, D), jnp.float32),
            pltpu.VMEM((1 + K, PW), jnp.float32),
            pltpu.SemaphoreType.DMA,
            pltpu.SemaphoreType.DMA,
            pltpu.SemaphoreType.DMA,
        ],
        compiler_params=pltpu.CompilerParams(
            needs_layout_passes=False, use_tc_tiling_on_sc=False),
    )
    def body(idx_hbm, uw_hbm, vw_hbm, out_hbm, idx_v, ru, rv, rn, sbuf, semu, semv, semn):
        wid = lax.axis_index("s") * NC + lax.axis_index("c")
        pltpu.sync_copy(idx_hbm.at[wid], idx_v)
        iota = lax.iota(jnp.int32, L)
        for c in range(NCH):
            cu = pltpu.async_copy(uw_hbm.at[idx_v.at[pl.ds(c * CH, CH)]], ru, semu)
            cv = pltpu.async_copy(vw_hbm.at[idx_v.at[pl.ds(PW + c * CH, CH)]], rv, semv)
            cns = [
                pltpu.async_copy(
                    vw_hbm.at[idx_v.at[pl.ds((2 + k) * PW + c * CH, CH)]],
                    rn.at[pl.ds(k * CH, CH)], semn)
                for k in range(K)
            ]
            cu.wait()
            cv.wait()
            for cn in cns:
                cn.wait()
            for g in range(G):
                rows = g * L + iota
                zero = jnp.zeros((L,), jnp.float32)

                def dbody(dd, accs, rows=rows):
                    dvec = jnp.zeros((L,), jnp.int32) + dd
                    u = plsc.load_gather(ru, [rows, dvec])
                    v = plsc.load_gather(rv, [rows, dvec])
                    s0 = accs[0] + u * v
                    ss = [
                        accs[1 + k]
                        + u * plsc.load_gather(rn, [k * CH + rows, dvec])
                        for k in range(K)
                    ]
                    return (s0, *ss)

                accs = lax.fori_loop(0, D, dbody, (zero,) * (1 + K))
                off = c * CH + g * L
                sbuf[0, pl.ds(off, L)] = -accs[0]
                for k in range(K):
                    sbuf[1 + k, pl.ds(off, L)] = accs[1 + k]
        pltpu.sync_copy(sbuf, out_hbm.at[wid])

    return body(idx_all, u_weight, v_weight)


def _loss_body(x_ref, o_ref):
    x = jnp.clip(x_ref[...], -10.0, 10.0)
    o_ref[...] = (jnp.sum(jnp.log1p(jnp.exp(x))) * (1.0 / B)).reshape(1, 1)


def kernel(pos_u, pos_v, neg_v, u_weight, v_weight):
    idx_all = jnp.concatenate(
        [pos_u[None, :], pos_v[None, :], neg_v.T.astype(jnp.int32)], axis=0)
    idx_all = idx_all.reshape(2 + K, NW, PW).transpose(1, 0, 2).reshape(NW, (2 + K) * PW)
    scores = _sc_scores(idx_all, u_weight, v_weight)
    loss = pl.pallas_call(
        _loss_body,
        out_shape=jax.ShapeDtypeStruct((1, 1), jnp.float32),
    )(scores.reshape(NW * (1 + K) * PW // 2048, 2048))
    return loss[0, 0]


# speedup vs baseline: 1.5915x; 1.0001x over previous
import functools

import jax
import jax.numpy as jnp
from jax import lax
from jax.experimental import pallas as pl
from jax.experimental.pallas import tpu as pltpu
from jax.experimental.pallas import tpu_sc as plsc

B = 16384
D = 64
K = 5
NODE = 1000000
NC = 2
NS = 16
NW = NC * NS
PW = B // NW
CH = 128
NCH = PW // CH
L = 16
G = CH // L


def _sc_scores(idx_all, u_weight, v_weight):
    mesh = plsc.VectorSubcoreMesh(core_axis_name="c", subcore_axis_name="s")

    @functools.partial(
        pl.kernel,
        out_type=jax.ShapeDtypeStruct((NW, 1 + K, PW), jnp.float32),
        mesh=mesh,
        scratch_types=[
            pltpu.VMEM(((2 + K) * PW,), jnp.int32),
            pltpu.VMEM((CH, D), jnp.float32),
            pltpu.VMEM((CH, D), jnp.float32),
            pltpu.VMEM((K * CH, D), jnp.float32),
            pltpu.VMEM((1 + K, PW), jnp.float32),
            pltpu.SemaphoreType.DMA,
            pltpu.SemaphoreType.DMA,
            pltpu.SemaphoreType.DMA,
        ],
        compiler_params=pltpu.CompilerParams(
            needs_layout_passes=False, use_tc_tiling_on_sc=False),
    )
    def body(idx_hbm, uw_hbm, vw_hbm, out_hbm, idx_v, ru, rv, rn, sbuf, semu, semv, semn):
        wid = lax.axis_index("s") * NC + lax.axis_index("c")
        pltpu.sync_copy(idx_hbm.at[wid], idx_v)
        iota = lax.iota(jnp.int32, L)
        for c in range(NCH):
            cu = pltpu.async_copy(uw_hbm.at[idx_v.at[pl.ds(c * CH, CH)]], ru, semu)
            cv = pltpu.async_copy(vw_hbm.at[idx_v.at[pl.ds(PW + c * CH, CH)]], rv, semv)
            cns = [
                pltpu.async_copy(
                    vw_hbm.at[idx_v.at[pl.ds((2 + k) * PW + c * CH, CH)]],
                    rn.at[pl.ds(k * CH, CH)], semn)
                for k in range(K)
            ]
            cu.wait()
            cv.wait()
            for cn in cns:
                cn.wait()
            for g in range(G):
                rows = g * L + iota
                zero = jnp.zeros((L,), jnp.float32)

                def dbody(dd, accs, rows=rows):
                    dvec = jnp.zeros((L,), jnp.int32) + dd
                    u = plsc.load_gather(ru, [rows, dvec])
                    v = plsc.load_gather(rv, [rows, dvec])
                    s0 = accs[0] + u * v
                    ss = [
                        accs[1 + k]
                        + u * plsc.load_gather(rn, [k * CH + rows, dvec])
                        for k in range(K)
                    ]
                    return (s0, *ss)

                accs = lax.fori_loop(0, D, dbody, (zero,) * (1 + K))
                off = c * CH + g * L
                sbuf[0, pl.ds(off, L)] = -accs[0]
                for k in range(K):
                    sbuf[1 + k, pl.ds(off, L)] = accs[1 + k]
        pltpu.sync_copy(sbuf, out_hbm.at[wid])

    return body(idx_all, u_weight, v_weight)


def _loss_body(x_ref, o_ref):
    x = jnp.clip(x_ref[...], -10.0, 10.0)
    o_ref[...] = (jnp.sum(jnp.log1p(jnp.exp(x))) * (1.0 / B)).reshape(1, 1)


def kernel(pos_u, pos_v, neg_v, u_weight, v_weight):
    idx_all = jnp.concatenate(
        [pos_u[None, :], pos_v[None, :], neg_v.T.astype(jnp.int32)], axis=0)
    idx_all = idx_all.reshape(2 + K, NW, PW).transpose(1, 0, 2).reshape(NW, (2 + K) * PW)
    u_lin = jax.lax.optimization_barrier(u_weight.reshape(NODE * D)).reshape(NODE, D)
    v_lin = jax.lax.optimization_barrier(v_weight.reshape(NODE * D)).reshape(NODE, D)
    scores = _sc_scores(idx_all, u_lin, v_lin)
    loss = pl.pallas_call(
        _loss_body,
        out_shape=jax.ShapeDtypeStruct((1, 1), jnp.float32),
    )(scores.reshape(NW * (1 + K) * PW // 2048, 2048))
    return loss[0, 0]


# speedup vs baseline: 1.8541x; 1.1650x over previous
import functools

import jax
import jax.numpy as jnp
from jax import lax
from jax.experimental import pallas as pl
from jax.experimental.pallas import tpu as pltpu
from jax.experimental.pallas import tpu_sc as plsc

B = 16384
D = 64
K = 5
NODE = 1000000
NP = NODE // 2
NC = 2
NS = 16
NW = NC * NS
PW = B // NW
CH = 64
NCH = PW // CH
L = 16
G = CH // L

TCW = 1024
TGRID = -(-NODE // (2 * TCW))
NPAD = TGRID * TCW


def _pack_body(x1_ref, x2_ref, o_ref):
    y1 = jnp.transpose(x1_ref[...])
    y2 = jnp.transpose(x2_ref[...])
    o_ref[...] = jnp.concatenate([y1, y2], axis=1)


def _pack_pairs(table_t):
    return pl.pallas_call(
        _pack_body,
        grid=(TGRID,),
        in_specs=[
            pl.BlockSpec((D, TCW), lambda j: (0, 2 * j)),
            pl.BlockSpec((D, TCW),
                         lambda j: (0, jnp.minimum(2 * j + 1, 2 * TGRID - 2))),
        ],
        out_specs=pl.BlockSpec((TCW, 2 * D), lambda j: (j, 0)),
        out_shape=jax.ShapeDtypeStruct((NPAD, 2 * D), jnp.float32),
    )(table_t, table_t)


def _sc_scores(idx_all, u_pack, v_pack):
    mesh = plsc.VectorSubcoreMesh(core_axis_name="c", subcore_axis_name="s")

    @functools.partial(
        pl.kernel,
        out_type=jax.ShapeDtypeStruct((NW, 1 + K, PW), jnp.float32),
        mesh=mesh,
        scratch_types=[
            pltpu.VMEM(((2 + K) * PW,), jnp.int32),
            pltpu.VMEM(((2 + K) * PW,), jnp.int32),
            pltpu.VMEM((CH, 2 * D), jnp.float32),
            pltpu.VMEM((CH, 2 * D), jnp.float32),
            pltpu.VMEM((K * CH, 2 * D), jnp.float32),
            pltpu.VMEM((1 + K, PW), jnp.float32),
            pltpu.SemaphoreType.DMA,
            pltpu.SemaphoreType.DMA,
            pltpu.SemaphoreType.DMA,
        ],
        compiler_params=pltpu.CompilerParams(
            needs_layout_passes=False, use_tc_tiling_on_sc=False),
    )
    def body(idx_hbm, uw_hbm, vw_hbm, out_hbm, idx_v, idxp_v, ru, rv, rn, sbuf,
             semu, semv, semn):
        wid = lax.axis_index("s") * NC + lax.axis_index("c")
        pltpu.sync_copy(idx_hbm.at[wid], idx_v)
        for i in range((2 + K) * PW // L):
            raw = idx_v[pl.ds(i * L, L)]
            idxp_v[pl.ds(i * L, L)] = ((raw >> 11) << 10) | (raw & 1023)
        iota = lax.iota(jnp.int32, L)
        for c in range(NCH):
            cu = pltpu.async_copy(
                uw_hbm.at[idxp_v.at[pl.ds(c * CH, CH)]], ru, semu)
            cv = pltpu.async_copy(
                vw_hbm.at[idxp_v.at[pl.ds(PW + c * CH, CH)]], rv, semv)
            cns = [
                pltpu.async_copy(
                    vw_hbm.at[idxp_v.at[pl.ds((2 + k) * PW + c * CH, CH)]],
                    rn.at[pl.ds(k * CH, CH)], semn)
                for k in range(K)
            ]
            cu.wait()
            cv.wait()
            for cn in cns:
                cn.wait()
            for g in range(G):
                rows = g * L + iota
                off = c * CH + g * L
                hu = ((idx_v[pl.ds(off, L)] >> 10) & 1) * D
                hv = ((idx_v[pl.ds(PW + off, L)] >> 10) & 1) * D
                hn = [((idx_v[pl.ds((2 + k) * PW + off, L)] >> 10) & 1) * D
                      for k in range(K)]
                zero = jnp.zeros((L,), jnp.float32)

                def dbody(dd, accs, rows=rows, hu=hu, hv=hv, hn=hn):
                    dvec = jnp.zeros((L,), jnp.int32) + dd
                    u = plsc.load_gather(ru, [rows, hu + dvec])
                    v = plsc.load_gather(rv, [rows, hv + dvec])
                    s0 = accs[0] + u * v
                    ss = [
                        accs[1 + k]
                        + u * plsc.load_gather(rn, [k * CH + rows, hn[k] + dvec])
                        for k in range(K)
                    ]
                    return (s0, *ss)

                accs = lax.fori_loop(0, D, dbody, (zero,) * (1 + K))
                sbuf[0, pl.ds(off, L)] = -accs[0]
                for k in range(K):
                    sbuf[1 + k, pl.ds(off, L)] = accs[1 + k]
        pltpu.sync_copy(sbuf, out_hbm.at[wid])

    return body(idx_all, u_pack, v_pack)


def _loss_body(x_ref, o_ref):
    x = jnp.clip(x_ref[...], -10.0, 10.0)
    o_ref[...] = (jnp.sum(jnp.log1p(jnp.exp(x))) * (1.0 / B)).reshape(1, 1)


def kernel(pos_u, pos_v, neg_v, u_weight, v_weight):
    idx_all = jnp.concatenate(
        [pos_u[None, :], pos_v[None, :], neg_v.T.astype(jnp.int32)], axis=0)
    idx_all = idx_all.reshape(2 + K, NW, PW).transpose(1, 0, 2).reshape(
        NW, (2 + K) * PW)
    u_pack = _pack_pairs(u_weight.T)
    v_pack = _pack_pairs(v_weight.T)
    scores = _sc_scores(idx_all, u_pack, v_pack)
    loss = pl.pallas_call(
        _loss_body,
        out_shape=jax.ShapeDtypeStruct((1, 1), jnp.float32),
    )(scores.reshape(NW * (1 + K) * PW // 2048, 2048))
    return loss[0, 0]


# speedup vs baseline: 2.7639x; 1.4907x over previous
import functools

import jax
import jax.numpy as jnp
from jax import lax
from jax.experimental import pallas as pl
from jax.experimental.pallas import tpu as pltpu
from jax.experimental.pallas import tpu_sc as plsc

B = 16384
D = 64
K = 5
NODE = 1000000
NP = NODE // 2
NC = 2
NS = 16
NW = NC * NS
PW = B // NW
CH = 64
NCH = PW // CH
L = 16
G = CH // L

TCW = 2048
TGRID = -(-NODE // (2 * TCW))
NPAD = TGRID * TCW
HSH = TCW.bit_length() - 1


def _pack_body(x_ref, o_ref):
    x = x_ref[...]
    xcat = jnp.concatenate([x[:, :TCW], x[:, TCW:]], axis=0)
    ident = (lax.broadcasted_iota(jnp.int32, (2 * D, 2 * D), 0)
             == lax.broadcasted_iota(jnp.int32, (2 * D, 2 * D), 1)
             ).astype(jnp.float32)
    dn = (((0,), (0,)), ((), ()))
    o_ref[...] = lax.dot_general(xcat, ident, dn,
                                 preferred_element_type=jnp.float32)


def _pack_pairs(table_t):
    return pl.pallas_call(
        _pack_body,
        grid=(TGRID,),
        in_specs=[pl.BlockSpec((D, 2 * TCW), lambda j: (0, j))],
        out_specs=pl.BlockSpec((TCW, 2 * D), lambda j: (j, 0)),
        out_shape=jax.ShapeDtypeStruct((NPAD, 2 * D), jnp.float32),
    )(table_t)


def _sc_scores(idx_all, u_pack, v_pack):
    mesh = plsc.VectorSubcoreMesh(core_axis_name="c", subcore_axis_name="s")

    @functools.partial(
        pl.kernel,
        out_type=jax.ShapeDtypeStruct((NW, 1 + K, PW), jnp.float32),
        mesh=mesh,
        scratch_types=[
            pltpu.VMEM(((2 + K) * PW,), jnp.int32),
            pltpu.VMEM(((2 + K) * PW,), jnp.int32),
            pltpu.VMEM((CH, 2 * D), jnp.float32),
            pltpu.VMEM((CH, 2 * D), jnp.float32),
            pltpu.VMEM((K * CH, 2 * D), jnp.float32),
            pltpu.VMEM((1 + K, PW), jnp.float32),
            pltpu.SemaphoreType.DMA,
            pltpu.SemaphoreType.DMA,
            pltpu.SemaphoreType.DMA,
        ],
        compiler_params=pltpu.CompilerParams(
            needs_layout_passes=False, use_tc_tiling_on_sc=False),
    )
    def body(idx_hbm, uw_hbm, vw_hbm, out_hbm, idx_v, idxp_v, ru, rv, rn, sbuf,
             semu, semv, semn):
        wid = lax.axis_index("s") * NC + lax.axis_index("c")
        pltpu.sync_copy(idx_hbm.at[wid], idx_v)
        for i in range((2 + K) * PW // L):
            raw = idx_v[pl.ds(i * L, L)]
            idxp_v[pl.ds(i * L, L)] = (
                ((raw >> (HSH + 1)) << HSH) | (raw & (TCW - 1)))
        iota = lax.iota(jnp.int32, L)
        for c in range(NCH):
            cu = pltpu.async_copy(
                uw_hbm.at[idxp_v.at[pl.ds(c * CH, CH)]], ru, semu)
            cv = pltpu.async_copy(
                vw_hbm.at[idxp_v.at[pl.ds(PW + c * CH, CH)]], rv, semv)
            cns = [
                pltpu.async_copy(
                    vw_hbm.at[idxp_v.at[pl.ds((2 + k) * PW + c * CH, CH)]],
                    rn.at[pl.ds(k * CH, CH)], semn)
                for k in range(K)
            ]
            cu.wait()
            cv.wait()
            for cn in cns:
                cn.wait()
            for g in range(G):
                rows = g * L + iota
                off = c * CH + g * L
                hu = ((idx_v[pl.ds(off, L)] >> HSH) & 1) * D
                hv = ((idx_v[pl.ds(PW + off, L)] >> HSH) & 1) * D
                hn = [((idx_v[pl.ds((2 + k) * PW + off, L)] >> HSH) & 1) * D
                      for k in range(K)]
                zero = jnp.zeros((L,), jnp.float32)

                def dbody(dd, accs, rows=rows, hu=hu, hv=hv, hn=hn):
                    dvec = jnp.zeros((L,), jnp.int32) + dd
                    u = plsc.load_gather(ru, [rows, hu + dvec])
                    v = plsc.load_gather(rv, [rows, hv + dvec])
                    s0 = accs[0] + u * v
                    ss = [
                        accs[1 + k]
                        + u * plsc.load_gather(rn, [k * CH + rows, hn[k] + dvec])
                        for k in range(K)
                    ]
                    return (s0, *ss)

                accs = lax.fori_loop(0, D, dbody, (zero,) * (1 + K))
                sbuf[0, pl.ds(off, L)] = -accs[0]
                for k in range(K):
                    sbuf[1 + k, pl.ds(off, L)] = accs[1 + k]
        pltpu.sync_copy(sbuf, out_hbm.at[wid])

    return body(idx_all, u_pack, v_pack)


def _loss_body(x_ref, o_ref):
    x = jnp.clip(x_ref[...], -10.0, 10.0)
    o_ref[...] = (jnp.sum(jnp.log1p(jnp.exp(x))) * (1.0 / B)).reshape(1, 1)


def kernel(pos_u, pos_v, neg_v, u_weight, v_weight):
    idx_all = jnp.concatenate(
        [pos_u[None, :], pos_v[None, :], neg_v.T.astype(jnp.int32)], axis=0)
    idx_all = idx_all.reshape(2 + K, NW, PW).transpose(1, 0, 2).reshape(
        NW, (2 + K) * PW)
    u_pack = _pack_pairs(u_weight.T)
    v_pack = _pack_pairs(v_weight.T)
    scores = _sc_scores(idx_all, u_pack, v_pack)
    loss = pl.pallas_call(
        _loss_body,
        out_shape=jax.ShapeDtypeStruct((1, 1), jnp.float32),
    )(scores.reshape(NW * (1 + K) * PW // 2048, 2048))
    return loss[0, 0]


# speedup vs baseline: 3.5270x; 1.2761x over previous
import functools

import jax
import jax.numpy as jnp
from jax import lax
from jax.experimental import pallas as pl
from jax.experimental.pallas import tpu as pltpu
from jax.experimental.pallas import tpu_sc as plsc

B = 16384
D = 64
K = 5
NODE = 1000000
NP = NODE // 2
NC = 2
NS = 16
NW = NC * NS
PW = B // NW
CH = 64
NCH = PW // CH
L = 16
G = CH // L

TCW = 4096
TGRID = -(-NODE // (2 * TCW))
NPAD = TGRID * TCW
HSH = TCW.bit_length() - 1


def _pack_body(x_ref, o_ref):
    x = x_ref[...]
    xcat = jnp.concatenate([x[:, :TCW], x[:, TCW:]], axis=0)
    ident = (lax.broadcasted_iota(jnp.int32, (2 * D, 2 * D), 0)
             == lax.broadcasted_iota(jnp.int32, (2 * D, 2 * D), 1)
             ).astype(jnp.float32)
    dn = (((0,), (0,)), ((), ()))
    o_ref[...] = lax.dot_general(xcat, ident, dn,
                                 preferred_element_type=jnp.float32)


def _pack_pairs(table_t):
    return pl.pallas_call(
        _pack_body,
        grid=(TGRID,),
        in_specs=[pl.BlockSpec((D, 2 * TCW), lambda j: (0, j))],
        out_specs=pl.BlockSpec((TCW, 2 * D), lambda j: (j, 0)),
        out_shape=jax.ShapeDtypeStruct((NPAD, 2 * D), jnp.float32),
    )(table_t)


def _sc_scores(idx_all, u_pack, v_pack):
    mesh = plsc.VectorSubcoreMesh(core_axis_name="c", subcore_axis_name="s")

    @functools.partial(
        pl.kernel,
        out_type=jax.ShapeDtypeStruct((NW, 1 + K, PW), jnp.float32),
        mesh=mesh,
        scratch_types=[
            pltpu.VMEM(((2 + K) * PW,), jnp.int32),
            pltpu.VMEM(((2 + K) * PW,), jnp.int32),
            pltpu.VMEM((CH, 2 * D), jnp.float32),
            pltpu.VMEM((CH, 2 * D), jnp.float32),
            pltpu.VMEM((K * CH, 2 * D), jnp.float32),
            pltpu.VMEM((1 + K, PW), jnp.float32),
            pltpu.SemaphoreType.DMA,
            pltpu.SemaphoreType.DMA,
            pltpu.SemaphoreType.DMA,
        ],
        compiler_params=pltpu.CompilerParams(
            needs_layout_passes=False, use_tc_tiling_on_sc=False),
    )
    def body(idx_hbm, uw_hbm, vw_hbm, out_hbm, idx_v, idxp_v, ru, rv, rn, sbuf,
             semu, semv, semn):
        wid = lax.axis_index("s") * NC + lax.axis_index("c")
        pltpu.sync_copy(idx_hbm.at[wid], idx_v)
        for i in range((2 + K) * PW // L):
            raw = idx_v[pl.ds(i * L, L)]
            idxp_v[pl.ds(i * L, L)] = (
                ((raw >> (HSH + 1)) << HSH) | (raw & (TCW - 1)))
        iota = lax.iota(jnp.int32, L)
        for c in range(NCH):
            cu = pltpu.async_copy(
                uw_hbm.at[idxp_v.at[pl.ds(c * CH, CH)]], ru, semu)
            cv = pltpu.async_copy(
                vw_hbm.at[idxp_v.at[pl.ds(PW + c * CH, CH)]], rv, semv)
            cns = [
                pltpu.async_copy(
                    vw_hbm.at[idxp_v.at[pl.ds((2 + k) * PW + c * CH, CH)]],
                    rn.at[pl.ds(k * CH, CH)], semn)
                for k in range(K)
            ]
            cu.wait()
            cv.wait()
            for cn in cns:
                cn.wait()
            for g in range(G):
                rows = g * L + iota
                off = c * CH + g * L
                hu = ((idx_v[pl.ds(off, L)] >> HSH) & 1) * D
                hv = ((idx_v[pl.ds(PW + off, L)] >> HSH) & 1) * D
                hn = [((idx_v[pl.ds((2 + k) * PW + off, L)] >> HSH) & 1) * D
                      for k in range(K)]
                zero = jnp.zeros((L,), jnp.float32)

                def dbody(dd, accs, rows=rows, hu=hu, hv=hv, hn=hn):
                    dvec = jnp.zeros((L,), jnp.int32) + dd
                    u = plsc.load_gather(ru, [rows, hu + dvec])
                    v = plsc.load_gather(rv, [rows, hv + dvec])
                    s0 = accs[0] + u * v
                    ss = [
                        accs[1 + k]
                        + u * plsc.load_gather(rn, [k * CH + rows, hn[k] + dvec])
                        for k in range(K)
                    ]
                    return (s0, *ss)

                accs = lax.fori_loop(0, D, dbody, (zero,) * (1 + K))
                sbuf[0, pl.ds(off, L)] = -accs[0]
                for k in range(K):
                    sbuf[1 + k, pl.ds(off, L)] = accs[1 + k]
        pltpu.sync_copy(sbuf, out_hbm.at[wid])

    return body(idx_all, u_pack, v_pack)


def _loss_body(x_ref, o_ref):
    x = jnp.clip(x_ref[...], -10.0, 10.0)
    o_ref[...] = (jnp.sum(jnp.log1p(jnp.exp(x))) * (1.0 / B)).reshape(1, 1)


def kernel(pos_u, pos_v, neg_v, u_weight, v_weight):
    idx_all = jnp.concatenate(
        [pos_u[None, :], pos_v[None, :], neg_v.T.astype(jnp.int32)], axis=0)
    idx_all = idx_all.reshape(2 + K, NW, PW).transpose(1, 0, 2).reshape(
        NW, (2 + K) * PW)
    u_pack = _pack_pairs(u_weight.T)
    v_pack = _pack_pairs(v_weight.T)
    scores = _sc_scores(idx_all, u_pack, v_pack)
    loss = pl.pallas_call(
        _loss_body,
        out_shape=jax.ShapeDtypeStruct((1, 1), jnp.float32),
    )(scores.reshape(NW * (1 + K) * PW // 2048, 2048))
    return loss[0, 0]


# speedup vs baseline: 3.9051x; 1.1072x over previous
import functools

import jax
import jax.numpy as jnp
from jax import lax
from jax.experimental import pallas as pl
from jax.experimental.pallas import tpu as pltpu
from jax.experimental.pallas import tpu_sc as plsc

B = 16384
D = 64
K = 5
NODE = 1000000
NP = NODE // 2
NC = 2
NS = 16
NW = NC * NS
PW = B // NW
CH = 64
NCH = PW // CH
L = 16
G = CH // L

TCW = 8192
TGRID = -(-NODE // (2 * TCW))
NPAD = TGRID * TCW
HSH = TCW.bit_length() - 1


def _pack_body(x_ref, o_ref):
    x = x_ref[...]
    xcat = jnp.concatenate([x[:, :TCW], x[:, TCW:]], axis=0)
    ident = (lax.broadcasted_iota(jnp.int32, (2 * D, 2 * D), 0)
             == lax.broadcasted_iota(jnp.int32, (2 * D, 2 * D), 1)
             ).astype(jnp.float32)
    dn = (((0,), (0,)), ((), ()))
    o_ref[...] = lax.dot_general(xcat, ident, dn,
                                 preferred_element_type=jnp.float32)


def _pack_pairs(table_t):
    return pl.pallas_call(
        _pack_body,
        grid=(TGRID,),
        in_specs=[pl.BlockSpec((D, 2 * TCW), lambda j: (0, j))],
        out_specs=pl.BlockSpec((TCW, 2 * D), lambda j: (j, 0)),
        out_shape=jax.ShapeDtypeStruct((NPAD, 2 * D), jnp.float32),
    )(table_t)


def _sc_scores(idx_all, u_pack, v_pack):
    mesh = plsc.VectorSubcoreMesh(core_axis_name="c", subcore_axis_name="s")

    @functools.partial(
        pl.kernel,
        out_type=jax.ShapeDtypeStruct((NW, 1 + K, PW), jnp.float32),
        mesh=mesh,
        scratch_types=[
            pltpu.VMEM(((2 + K) * PW,), jnp.int32),
            pltpu.VMEM(((2 + K) * PW,), jnp.int32),
            pltpu.VMEM((CH, 2 * D), jnp.float32),
            pltpu.VMEM((CH, 2 * D), jnp.float32),
            pltpu.VMEM((K * CH, 2 * D), jnp.float32),
            pltpu.VMEM((1 + K, PW), jnp.float32),
            pltpu.SemaphoreType.DMA,
            pltpu.SemaphoreType.DMA,
            pltpu.SemaphoreType.DMA,
        ],
        compiler_params=pltpu.CompilerParams(
            needs_layout_passes=False, use_tc_tiling_on_sc=False),
    )
    def body(idx_hbm, uw_hbm, vw_hbm, out_hbm, idx_v, idxp_v, ru, rv, rn, sbuf,
             semu, semv, semn):
        wid = lax.axis_index("s") * NC + lax.axis_index("c")
        pltpu.sync_copy(idx_hbm.at[wid], idx_v)
        for i in range((2 + K) * PW // L):
            raw = idx_v[pl.ds(i * L, L)]
            idxp_v[pl.ds(i * L, L)] = (
                ((raw >> (HSH + 1)) << HSH) | (raw & (TCW - 1)))
        iota = lax.iota(jnp.int32, L)
        for c in range(NCH):
            cu = pltpu.async_copy(
                uw_hbm.at[idxp_v.at[pl.ds(c * CH, CH)]], ru, semu)
            cv = pltpu.async_copy(
                vw_hbm.at[idxp_v.at[pl.ds(PW + c * CH, CH)]], rv, semv)
            cns = [
                pltpu.async_copy(
                    vw_hbm.at[idxp_v.at[pl.ds((2 + k) * PW + c * CH, CH)]],
                    rn.at[pl.ds(k * CH, CH)], semn)
                for k in range(K)
            ]
            cu.wait()
            cv.wait()
            for cn in cns:
                cn.wait()
            for g in range(G):
                rows = g * L + iota
                off = c * CH + g * L
                hu = ((idx_v[pl.ds(off, L)] >> HSH) & 1) * D
                hv = ((idx_v[pl.ds(PW + off, L)] >> HSH) & 1) * D
                hn = [((idx_v[pl.ds((2 + k) * PW + off, L)] >> HSH) & 1) * D
                      for k in range(K)]
                zero = jnp.zeros((L,), jnp.float32)

                def dbody(dd, accs, rows=rows, hu=hu, hv=hv, hn=hn):
                    dvec = jnp.zeros((L,), jnp.int32) + dd
                    u = plsc.load_gather(ru, [rows, hu + dvec])
                    v = plsc.load_gather(rv, [rows, hv + dvec])
                    s0 = accs[0] + u * v
                    ss = [
                        accs[1 + k]
                        + u * plsc.load_gather(rn, [k * CH + rows, hn[k] + dvec])
                        for k in range(K)
                    ]
                    return (s0, *ss)

                accs = lax.fori_loop(0, D, dbody, (zero,) * (1 + K))
                sbuf[0, pl.ds(off, L)] = -accs[0]
                for k in range(K):
                    sbuf[1 + k, pl.ds(off, L)] = accs[1 + k]
        pltpu.sync_copy(sbuf, out_hbm.at[wid])

    return body(idx_all, u_pack, v_pack)


def _loss_body(x_ref, o_ref):
    x = jnp.clip(x_ref[...], -10.0, 10.0)
    o_ref[...] = (jnp.sum(jnp.log1p(jnp.exp(x))) * (1.0 / B)).reshape(1, 1)


def kernel(pos_u, pos_v, neg_v, u_weight, v_weight):
    idx_all = jnp.concatenate(
        [pos_u[None, :], pos_v[None, :], neg_v.T.astype(jnp.int32)], axis=0)
    idx_all = idx_all.reshape(2 + K, NW, PW).transpose(1, 0, 2).reshape(
        NW, (2 + K) * PW)
    u_pack = _pack_pairs(u_weight.T)
    v_pack = _pack_pairs(v_weight.T)
    scores = _sc_scores(idx_all, u_pack, v_pack)
    loss = pl.pallas_call(
        _loss_body,
        out_shape=jax.ShapeDtypeStruct((1, 1), jnp.float32),
    )(scores.reshape(NW * (1 + K) * PW // 2048, 2048))
    return loss[0, 0]


# speedup vs baseline: 3.9750x; 1.0179x over previous
import functools

import jax
import jax.numpy as jnp
from jax import lax
from jax.experimental import pallas as pl
from jax.experimental.pallas import tpu as pltpu
from jax.experimental.pallas import tpu_sc as plsc

B = 16384
D = 64
K = 5
NODE = 1000000
NP = NODE // 2
NC = 2
NS = 16
NW = NC * NS
PW = B // NW
CH = 64
NCH = PW // CH
L = 16
G = CH // L

TCW = 16384
TGRID = -(-NODE // (2 * TCW))
NPAD = TGRID * TCW
HSH = TCW.bit_length() - 1


def _pack_body(x_ref, o_ref):
    x = x_ref[...]
    xcat = jnp.concatenate([x[:, :TCW], x[:, TCW:]], axis=0)
    ident = (lax.broadcasted_iota(jnp.int32, (2 * D, 2 * D), 0)
             == lax.broadcasted_iota(jnp.int32, (2 * D, 2 * D), 1)
             ).astype(jnp.float32)
    dn = (((0,), (0,)), ((), ()))
    o_ref[...] = lax.dot_general(xcat, ident, dn,
                                 preferred_element_type=jnp.float32)


def _pack_pairs(table_t):
    return pl.pallas_call(
        _pack_body,
        grid=(TGRID,),
        in_specs=[pl.BlockSpec((D, 2 * TCW), lambda j: (0, j))],
        out_specs=pl.BlockSpec((TCW, 2 * D), lambda j: (j, 0)),
        out_shape=jax.ShapeDtypeStruct((NPAD, 2 * D), jnp.float32),
    )(table_t)


def _sc_scores(idx_all, u_pack, v_pack):
    mesh = plsc.VectorSubcoreMesh(core_axis_name="c", subcore_axis_name="s")

    @functools.partial(
        pl.kernel,
        out_type=jax.ShapeDtypeStruct((NW, 1 + K, PW), jnp.float32),
        mesh=mesh,
        scratch_types=[
            pltpu.VMEM(((2 + K) * PW,), jnp.int32),
            pltpu.VMEM(((2 + K) * PW,), jnp.int32),
            pltpu.VMEM((CH, 2 * D), jnp.float32),
            pltpu.VMEM((CH, 2 * D), jnp.float32),
            pltpu.VMEM((K * CH, 2 * D), jnp.float32),
            pltpu.VMEM((1 + K, PW), jnp.float32),
            pltpu.SemaphoreType.DMA,
            pltpu.SemaphoreType.DMA,
            pltpu.SemaphoreType.DMA,
        ],
        compiler_params=pltpu.CompilerParams(
            needs_layout_passes=False, use_tc_tiling_on_sc=False),
    )
    def body(idx_hbm, uw_hbm, vw_hbm, out_hbm, idx_v, idxp_v, ru, rv, rn, sbuf,
             semu, semv, semn):
        wid = lax.axis_index("s") * NC + lax.axis_index("c")
        pltpu.sync_copy(idx_hbm.at[wid], idx_v)
        for i in range((2 + K) * PW // L):
            raw = idx_v[pl.ds(i * L, L)]
            idxp_v[pl.ds(i * L, L)] = (
                ((raw >> (HSH + 1)) << HSH) | (raw & (TCW - 1)))
        iota = lax.iota(jnp.int32, L)
        for c in range(NCH):
            cu = pltpu.async_copy(
                uw_hbm.at[idxp_v.at[pl.ds(c * CH, CH)]], ru, semu)
            cv = pltpu.async_copy(
                vw_hbm.at[idxp_v.at[pl.ds(PW + c * CH, CH)]], rv, semv)
            cns = [
                pltpu.async_copy(
                    vw_hbm.at[idxp_v.at[pl.ds((2 + k) * PW + c * CH, CH)]],
                    rn.at[pl.ds(k * CH, CH)], semn)
                for k in range(K)
            ]
            cu.wait()
            cv.wait()
            for cn in cns:
                cn.wait()
            for g in range(G):
                rows = g * L + iota
                off = c * CH + g * L
                hu = ((idx_v[pl.ds(off, L)] >> HSH) & 1) * D
                hv = ((idx_v[pl.ds(PW + off, L)] >> HSH) & 1) * D
                hn = [((idx_v[pl.ds((2 + k) * PW + off, L)] >> HSH) & 1) * D
                      for k in range(K)]
                zero = jnp.zeros((L,), jnp.float32)

                def dbody(dd, accs, rows=rows, hu=hu, hv=hv, hn=hn):
                    dvec = jnp.zeros((L,), jnp.int32) + dd
                    u = plsc.load_gather(ru, [rows, hu + dvec])
                    v = plsc.load_gather(rv, [rows, hv + dvec])
                    s0 = accs[0] + u * v
                    ss = [
                        accs[1 + k]
                        + u * plsc.load_gather(rn, [k * CH + rows, hn[k] + dvec])
                        for k in range(K)
                    ]
                    return (s0, *ss)

                accs = lax.fori_loop(0, D, dbody, (zero,) * (1 + K))
                sbuf[0, pl.ds(off, L)] = -accs[0]
                for k in range(K):
                    sbuf[1 + k, pl.ds(off, L)] = accs[1 + k]
        pltpu.sync_copy(sbuf, out_hbm.at[wid])

    return body(idx_all, u_pack, v_pack)


def _loss_body(x_ref, o_ref):
    x = jnp.clip(x_ref[...], -10.0, 10.0)
    o_ref[...] = (jnp.sum(jnp.log1p(jnp.exp(x))) * (1.0 / B)).reshape(1, 1)


def kernel(pos_u, pos_v, neg_v, u_weight, v_weight):
    idx_all = jnp.concatenate(
        [pos_u[None, :], pos_v[None, :], neg_v.T.astype(jnp.int32)], axis=0)
    idx_all = idx_all.reshape(2 + K, NW, PW).transpose(1, 0, 2).reshape(
        NW, (2 + K) * PW)
    u_pack = _pack_pairs(u_weight.T)
    v_pack = _pack_pairs(v_weight.T)
    scores = _sc_scores(idx_all, u_pack, v_pack)
    loss = pl.pallas_call(
        _loss_body,
        out_shape=jax.ShapeDtypeStruct((1, 1), jnp.float32),
    )(scores.reshape(NW * (1 + K) * PW // 2048, 2048))
    return loss[0, 0]


# speedup vs baseline: 4.1589x; 1.0463x over previous
import functools

import jax
import jax.numpy as jnp
from jax import lax
from jax.experimental import pallas as pl
from jax.experimental.pallas import tpu as pltpu
from jax.experimental.pallas import tpu_sc as plsc

B = 16384
D = 64
K = 5
NODE = 1000000
NP = NODE // 2
NC = 2
NS = 16
NW = NC * NS
PW = B // NW
CH = 128
NCH = PW // CH
L = 16
G = CH // L

TCW = 16384
TGRID = -(-NODE // (2 * TCW))
NPAD = TGRID * TCW
HSH = TCW.bit_length() - 1


def _pack_body(x_ref, o_ref):
    x = x_ref[...]
    xcat = jnp.concatenate([x[:, :TCW], x[:, TCW:]], axis=0)
    ident = (lax.broadcasted_iota(jnp.int32, (2 * D, 2 * D), 0)
             == lax.broadcasted_iota(jnp.int32, (2 * D, 2 * D), 1)
             ).astype(jnp.float32)
    dn = (((0,), (0,)), ((), ()))
    o_ref[...] = lax.dot_general(xcat, ident, dn,
                                 preferred_element_type=jnp.float32)


def _pack_pairs(table_t):
    return pl.pallas_call(
        _pack_body,
        grid=(TGRID,),
        in_specs=[pl.BlockSpec((D, 2 * TCW), lambda j: (0, j))],
        out_specs=pl.BlockSpec((TCW, 2 * D), lambda j: (j, 0)),
        out_shape=jax.ShapeDtypeStruct((NPAD, 2 * D), jnp.float32),
    )(table_t)


def _sc_scores(idx_all, u_pack, v_pack):
    mesh = plsc.VectorSubcoreMesh(core_axis_name="c", subcore_axis_name="s")

    @functools.partial(
        pl.kernel,
        out_type=jax.ShapeDtypeStruct((NW, 1 + K, PW), jnp.float32),
        mesh=mesh,
        scratch_types=[
            pltpu.VMEM(((2 + K) * PW,), jnp.int32),
            pltpu.VMEM(((2 + K) * PW,), jnp.int32),
            pltpu.VMEM((2 * CH, D), jnp.float32),
            pltpu.VMEM((2 * CH, D), jnp.float32),
            pltpu.VMEM((2 * K * CH, D), jnp.float32),
            pltpu.VMEM((1 + K, PW), jnp.float32),
            pltpu.SemaphoreType.DMA,
            pltpu.SemaphoreType.DMA,
            pltpu.SemaphoreType.DMA,
            pltpu.SemaphoreType.DMA,
            pltpu.SemaphoreType.DMA,
            pltpu.SemaphoreType.DMA,
        ],
        compiler_params=pltpu.CompilerParams(
            needs_layout_passes=False, use_tc_tiling_on_sc=False),
    )
    def body(idx_hbm, uw_hbm, vw_hbm, out_hbm, idx_v, idxp_v, ru, rv, rn, sbuf,
             *sems):
        wid = lax.axis_index("s") * NC + lax.axis_index("c")
        pltpu.sync_copy(idx_hbm.at[wid], idx_v)
        for i in range((2 + K) * PW // L):
            raw = idx_v[pl.ds(i * L, L)]
            q = ((raw >> (HSH + 1)) << HSH) | (raw & (TCW - 1))
            idxp_v[pl.ds(i * L, L)] = (q << 1) | ((raw >> HSH) & 1)
        iota = lax.iota(jnp.int32, L)

        def issue(c, b):
            cu = pltpu.async_copy(
                uw_hbm.at[idxp_v.at[pl.ds(c * CH, CH)]],
                ru.at[pl.ds(b * CH, CH)], sems[b])
            cv = pltpu.async_copy(
                vw_hbm.at[idxp_v.at[pl.ds(PW + c * CH, CH)]],
                rv.at[pl.ds(b * CH, CH)], sems[2 + b])
            cns = [
                pltpu.async_copy(
                    vw_hbm.at[idxp_v.at[pl.ds((2 + k) * PW + c * CH, CH)]],
                    rn.at[pl.ds((b * K + k) * CH, CH)], sems[4 + b])
                for k in range(K)
            ]
            return [cu, cv] + cns

        pending = {0: issue(0, 0)}
        for c in range(NCH):
            b = c & 1
            if c + 1 < NCH:
                pending[c + 1] = issue(c + 1, 1 - b)
            for cp in pending.pop(c):
                cp.wait()
            for g in range(G):
                rows = b * K * CH + g * L + iota
                rowuv = b * CH + g * L + iota
                off = c * CH + g * L
                zero = jnp.zeros((L,), jnp.float32)

                def dbody(dd, accs, rows=rows, rowuv=rowuv):
                    dvec = jnp.zeros((L,), jnp.int32) + dd
                    u = plsc.load_gather(ru, [rowuv, dvec])
                    v = plsc.load_gather(rv, [rowuv, dvec])
                    s0 = accs[0] + u * v
                    ss = [
                        accs[1 + k]
                        + u * plsc.load_gather(rn, [k * CH + rows, dvec])
                        for k in range(K)
                    ]
                    return (s0, *ss)

                accs = lax.fori_loop(0, D, dbody, (zero,) * (1 + K))
                sbuf[0, pl.ds(off, L)] = -accs[0]
                for k in range(K):
                    sbuf[1 + k, pl.ds(off, L)] = accs[1 + k]
        pltpu.sync_copy(sbuf, out_hbm.at[wid])

    return body(idx_all, u_pack, v_pack)


def _loss_body(x_ref, o_ref):
    x = jnp.clip(x_ref[...], -10.0, 10.0)
    o_ref[...] = (jnp.sum(jnp.log1p(jnp.exp(x))) * (1.0 / B)).reshape(1, 1)


def kernel(pos_u, pos_v, neg_v, u_weight, v_weight):
    idx_all = jnp.concatenate(
        [pos_u[None, :], pos_v[None, :], neg_v.T.astype(jnp.int32)], axis=0)
    idx_all = idx_all.reshape(2 + K, NW, PW).transpose(1, 0, 2).reshape(
        NW, (2 + K) * PW)
    u_pack = _pack_pairs(u_weight.T).reshape(2 * NPAD, D)
    v_pack = _pack_pairs(v_weight.T).reshape(2 * NPAD, D)
    scores = _sc_scores(idx_all, u_pack, v_pack)
    loss = pl.pallas_call(
        _loss_body,
        out_shape=jax.ShapeDtypeStruct((1, 1), jnp.float32),
    )(scores.reshape(NW * (1 + K) * PW // 2048, 2048))
    return loss[0, 0]


# speedup vs baseline: 4.3113x; 1.0366x over previous
import functools

import jax
import jax.numpy as jnp
from jax import lax
from jax.experimental import pallas as pl
from jax.experimental.pallas import tpu as pltpu
from jax.experimental.pallas import tpu_sc as plsc

B = 16384
D = 64
K = 5
NODE = 1000000
NP = NODE // 2
NC = 2
NS = 16
NW = NC * NS
PW = B // NW
CH = 128
NCH = PW // CH
L = 16
G = CH // L

TCW = 16384
TGRID = -(-NODE // (2 * TCW))
NPAD = TGRID * TCW
HSH = TCW.bit_length() - 1


def _pack_body(x_ref, o_ref):
    x = x_ref[...]
    xcat = jnp.concatenate([x[:, :TCW], x[:, TCW:]], axis=0)
    ident = (lax.broadcasted_iota(jnp.int32, (2 * D, 2 * D), 0)
             == lax.broadcasted_iota(jnp.int32, (2 * D, 2 * D), 1)
             ).astype(jnp.float32)
    dn = (((0,), (0,)), ((), ()))
    o_ref[...] = lax.dot_general(xcat, ident, dn,
                                 preferred_element_type=jnp.float32)


def _pack_pairs(table_t):
    return pl.pallas_call(
        _pack_body,
        grid=(TGRID,),
        in_specs=[pl.BlockSpec((D, 2 * TCW), lambda j: (0, j))],
        out_specs=pl.BlockSpec((TCW, 2 * D), lambda j: (j, 0)),
        out_shape=jax.ShapeDtypeStruct((NPAD, 2 * D), jnp.float32),
    )(table_t)


def _sc_scores(idx_all, u_pack, v_pack):
    mesh = plsc.VectorSubcoreMesh(core_axis_name="c", subcore_axis_name="s")

    @functools.partial(
        pl.kernel,
        out_type=jax.ShapeDtypeStruct((NW, 1 + K, PW), jnp.float32),
        mesh=mesh,
        scratch_types=[
            pltpu.VMEM(((2 + K) * PW,), jnp.int32),
            pltpu.VMEM(((2 + K) * PW,), jnp.int32),
            pltpu.VMEM((2 * CH, D), jnp.float32),
            pltpu.VMEM((2 * CH, D), jnp.float32),
            pltpu.VMEM((2 * K * CH, D), jnp.float32),
            pltpu.VMEM((1 + K, PW), jnp.float32),
            pltpu.SemaphoreType.DMA,
            pltpu.SemaphoreType.DMA,
            pltpu.SemaphoreType.DMA,
            pltpu.SemaphoreType.DMA,
            pltpu.SemaphoreType.DMA,
            pltpu.SemaphoreType.DMA,
        ],
        compiler_params=pltpu.CompilerParams(
            needs_layout_passes=False, use_tc_tiling_on_sc=False),
    )
    def body(idx_hbm, uw_hbm, vw_hbm, out_hbm, idx_v, idxp_v, ru, rv, rn, sbuf,
             *sems):
        wid = lax.axis_index("s") * NC + lax.axis_index("c")
        pltpu.sync_copy(idx_hbm.at[wid], idx_v)
        for i in range((2 + K) * PW // L):
            raw = idx_v[pl.ds(i * L, L)]
            q = ((raw >> (HSH + 1)) << HSH) | (raw & (TCW - 1))
            idxp_v[pl.ds(i * L, L)] = (q << 1) | ((raw >> HSH) & 1)
        iota = lax.iota(jnp.int32, L)

        def issue(c, b):
            cu = pltpu.async_copy(
                uw_hbm.at[idxp_v.at[pl.ds(c * CH, CH)]],
                ru.at[pl.ds(b * CH, CH)], sems[b])
            cv = pltpu.async_copy(
                vw_hbm.at[idxp_v.at[pl.ds(PW + c * CH, CH)]],
                rv.at[pl.ds(b * CH, CH)], sems[2 + b])
            cns = [
                pltpu.async_copy(
                    vw_hbm.at[idxp_v.at[pl.ds((2 + k) * PW + c * CH, CH)]],
                    rn.at[pl.ds((b * K + k) * CH, CH)], sems[4 + b])
                for k in range(K)
            ]
            return [cu, cv] + cns

        pending = {0: issue(0, 0)}
        for c in range(NCH):
            b = c & 1
            if c + 1 < NCH:
                pending[c + 1] = issue(c + 1, 1 - b)
            for cp in pending.pop(c):
                cp.wait()
            for g in range(G):
                rows = b * K * CH + g * L + iota
                rowuv = b * CH + g * L + iota
                off = c * CH + g * L
                zero = jnp.zeros((L,), jnp.float32)

                def dbody(dd, accs, rows=rows, rowuv=rowuv):
                    base = jnp.zeros((L,), jnp.int32) + dd * 4
                    for t in range(4):
                        dvec = base + t
                        u = plsc.load_gather(ru, [rowuv, dvec])
                        v = plsc.load_gather(rv, [rowuv, dvec])
                        s0 = accs[0] + u * v
                        ss = [
                            accs[1 + k]
                            + u * plsc.load_gather(rn, [k * CH + rows, dvec])
                            for k in range(K)
                        ]
                        accs = (s0, *ss)
                    return accs

                accs = lax.fori_loop(0, D // 4, dbody, (zero,) * (1 + K))
                sbuf[0, pl.ds(off, L)] = -accs[0]
                for k in range(K):
                    sbuf[1 + k, pl.ds(off, L)] = accs[1 + k]
        pltpu.sync_copy(sbuf, out_hbm.at[wid])

    return body(idx_all, u_pack, v_pack)


def _loss_body(x_ref, o_ref):
    x = jnp.clip(x_ref[...], -10.0, 10.0)
    o_ref[...] = (jnp.sum(jnp.log1p(jnp.exp(x))) * (1.0 / B)).reshape(1, 1)


def kernel(pos_u, pos_v, neg_v, u_weight, v_weight):
    idx_all = jnp.concatenate(
        [pos_u[None, :], pos_v[None, :], neg_v.T.astype(jnp.int32)], axis=0)
    idx_all = idx_all.reshape(2 + K, NW, PW).transpose(1, 0, 2).reshape(
        NW, (2 + K) * PW)
    u_pack = _pack_pairs(u_weight.T).reshape(2 * NPAD, D)
    v_pack = _pack_pairs(v_weight.T).reshape(2 * NPAD, D)
    scores = _sc_scores(idx_all, u_pack, v_pack)
    loss = pl.pallas_call(
        _loss_body,
        out_shape=jax.ShapeDtypeStruct((1, 1), jnp.float32),
    )(scores.reshape(NW * (1 + K) * PW // 2048, 2048))
    return loss[0, 0]
